# baseline (speedup 1.0000x reference)
import functools

import jax
import jax.numpy as jnp
from jax import lax
from jax.experimental import pallas as pl
from jax.experimental.pallas import tpu as pltpu
from jax.experimental.pallas import tpu_sc as plsc

VOCAB = 1000000
DIM = 64
REC = 128
BATCH = 16384

TBLK = 16384
TGRID = 31
S = TGRID * TBLK
LAST_VALID_BLK = (VOCAB - 1) // TBLK

NUM_CORES = 2
NUM_SUBCORES = 16
NUM_WORKERS = NUM_CORES * NUM_SUBCORES
BPW = BATCH // NUM_WORKERS
CHUNK = 128
CHUNKS = BPW // CHUNK
GROUPS = CHUNK // 16


def _pack_body(left_ref, right_ref, out_ref):
    out_ref[...] = jnp.concatenate(
        [jnp.swapaxes(left_ref[...], 0, 1),
         jnp.swapaxes(right_ref[...], 0, 1)], axis=1)


def _pack(table_t):
    return pl.pallas_call(
        _pack_body,
        grid=(TGRID,),
        in_specs=[
            pl.BlockSpec((DIM, TBLK), lambda j: (0, j)),
            pl.BlockSpec((DIM, TBLK),
                         lambda j: (0, jnp.minimum(j + TGRID, LAST_VALID_BLK))),
        ],
        out_specs=pl.BlockSpec((TBLK, REC), lambda j: (j, 0)),
        out_shape=jax.ShapeDtypeStruct((S, REC), jnp.float32),
    )(table_t, table_t)


@functools.partial(
    pl.kernel,
    out_type=jax.ShapeDtypeStruct((BATCH,), jnp.float32),
    mesh=plsc.VectorSubcoreMesh(core_axis_name="c", subcore_axis_name="s"),
    compiler_params=pltpu.CompilerParams(use_tc_tiling_on_sc=True),
    scratch_types=[
        pltpu.VMEM((CHUNKS, CHUNK), jnp.int32),
        pltpu.VMEM((CHUNKS, CHUNK), jnp.int32),
        pltpu.VMEM((CHUNKS, CHUNK), jnp.float32),
        pltpu.VMEM((CHUNKS, CHUNK), jnp.float32),
        pltpu.VMEM((2, CHUNK, REC), jnp.float32),
        pltpu.VMEM((2, CHUNK, REC), jnp.float32),
        pltpu.VMEM((BPW,), jnp.float32),
        pltpu.SemaphoreType.DMA,
    ],
)
def _w2v_dot(pv, pu, cq, xq, ch, xh, out_hbm,
             cq_v, xq_v, ch_v, xh_v, vbuf, ubuf, out_v, sem):
    wid = lax.axis_index("s") * NUM_CORES + lax.axis_index("c")
    base = wid * BPW

    pltpu.sync_copy(cq.at[wid], cq_v)
    pltpu.sync_copy(xq.at[wid], xq_v)
    pltpu.sync_copy(ch.at[wid], ch_v)
    pltpu.sync_copy(xh.at[wid], xh_v)

    def fire(c):
        slot = c % 2
        return (
            pltpu.async_copy(pv.at[cq_v.at[c]], vbuf.at[slot], sem),
            pltpu.async_copy(pu.at[xq_v.at[c]], ubuf.at[slot], sem),
        )

    lanes = lax.iota(jnp.int32, 16)
    pending = fire(0)
    for c in range(CHUNKS):
        nxt = fire(c + 1) if c + 1 < CHUNKS else None
        for cp in pending:
            cp.wait()
        slot = c % 2

        def group(g, carry):
            r0 = g * 16
            hv16 = ch_v[c, pl.ds(r0, 16)]
            hu16 = xh_v[c, pl.ds(r0, 16)]
            acc = jnp.zeros((16,), jnp.float32)
            for j in range(16):
                r = r0 + j
                jj = jnp.full((16,), j, jnp.int32)
                hv = hv16.at[jj].get(mode="promise_in_bounds")
                hu = hu16.at[jj].get(mode="promise_in_bounds")
                p = jnp.zeros((16,), jnp.float32)
                for k in range(4):
                    alo = vbuf[slot, r, pl.ds(k * 16, 16)]
                    a = alo + hv * (vbuf[slot, r, pl.ds(DIM + k * 16, 16)] - alo)
                    blo = ubuf[slot, r, pl.ds(k * 16, 16)]
                    b = blo + hu * (ubuf[slot, r, pl.ds(DIM + k * 16, 16)] - blo)
                    p = p + a * b
                for sh in (1, 2, 4, 8):
                    p = p + p.at[lanes ^ sh].get(mode="promise_in_bounds")
                acc = jnp.where(lanes == j, p, acc)
            out_v[pl.ds(c * CHUNK + r0, 16)] = acc
            return carry

        lax.fori_loop(0, GROUPS, group, 0)
        pending = nxt

    pltpu.sync_copy(out_v, out_hbm.at[pl.ds(base, BPW)])


def kernel(center_idx, context_idx, in_embed, out_embed):
    pv = _pack(in_embed.T)
    pu = _pack(out_embed.T)
    cidx = center_idx.astype(jnp.int32)
    xidx = context_idx.astype(jnp.int32)
    cq = jnp.where(cidx >= S, cidx - S, cidx).reshape(NUM_WORKERS, CHUNKS, CHUNK)
    xq = jnp.where(xidx >= S, xidx - S, xidx).reshape(NUM_WORKERS, CHUNKS, CHUNK)
    ch = (cidx >= S).astype(jnp.float32).reshape(NUM_WORKERS, CHUNKS, CHUNK)
    xh = (xidx >= S).astype(jnp.float32).reshape(NUM_WORKERS, CHUNKS, CHUNK)
    return _w2v_dot(pv, pu, cq, xq, ch, xh)


# speedup vs baseline: 1.0029x; 1.0029x over previous
import functools

import jax
import jax.numpy as jnp
from jax import lax
from jax.experimental import pallas as pl
from jax.experimental.pallas import tpu as pltpu
from jax.experimental.pallas import tpu_sc as plsc

VOCAB = 1000000
DIM = 64
REC = 128
BATCH = 16384

TBLK = 16384
TGRID = 31
S = TGRID * TBLK
LAST_VALID_BLK = (VOCAB - 1) // TBLK

NUM_CORES = 2
NUM_SUBCORES = 16
NUM_WORKERS = NUM_CORES * NUM_SUBCORES
BPW = BATCH // NUM_WORKERS
CHUNK = 128
CHUNKS = BPW // CHUNK
GROUPS = CHUNK // 16


def _pack_body(left_ref, right_ref, out_ref):
    out_ref[:, 0:DIM] = jnp.swapaxes(left_ref[...], 0, 1)
    out_ref[:, DIM:REC] = jnp.swapaxes(right_ref[...], 0, 1)


def _pack(table_t):
    return pl.pallas_call(
        _pack_body,
        grid=(TGRID,),
        in_specs=[
            pl.BlockSpec((DIM, TBLK), lambda j: (0, j)),
            pl.BlockSpec((DIM, TBLK),
                         lambda j: (0, jnp.minimum(j + TGRID, LAST_VALID_BLK))),
        ],
        out_specs=pl.BlockSpec((TBLK, REC), lambda j: (j, 0)),
        out_shape=jax.ShapeDtypeStruct((S, REC), jnp.float32),
    )(table_t, table_t)


@functools.partial(
    pl.kernel,
    out_type=jax.ShapeDtypeStruct((BATCH,), jnp.float32),
    mesh=plsc.VectorSubcoreMesh(core_axis_name="c", subcore_axis_name="s"),
    compiler_params=pltpu.CompilerParams(use_tc_tiling_on_sc=True),
    scratch_types=[
        pltpu.VMEM((CHUNKS, CHUNK), jnp.int32),
        pltpu.VMEM((CHUNKS, CHUNK), jnp.int32),
        pltpu.VMEM((CHUNKS, CHUNK), jnp.float32),
        pltpu.VMEM((CHUNKS, CHUNK), jnp.float32),
        pltpu.VMEM((2, CHUNK, REC), jnp.float32),
        pltpu.VMEM((2, CHUNK, REC), jnp.float32),
        pltpu.VMEM((BPW,), jnp.float32),
        pltpu.SemaphoreType.DMA,
    ],
)
def _w2v_dot(pv, pu, cq, xq, ch, xh, out_hbm,
             cq_v, xq_v, ch_v, xh_v, vbuf, ubuf, out_v, sem):
    wid = lax.axis_index("s") * NUM_CORES + lax.axis_index("c")
    base = wid * BPW

    pltpu.sync_copy(cq.at[wid], cq_v)
    pltpu.sync_copy(xq.at[wid], xq_v)
    pltpu.sync_copy(ch.at[wid], ch_v)
    pltpu.sync_copy(xh.at[wid], xh_v)

    def fire(c):
        slot = c % 2
        return (
            pltpu.async_copy(pv.at[cq_v.at[c]], vbuf.at[slot], sem),
            pltpu.async_copy(pu.at[xq_v.at[c]], ubuf.at[slot], sem),
        )

    lanes = lax.iota(jnp.int32, 16)
    pending = fire(0)
    for c in range(CHUNKS):
        nxt = fire(c + 1) if c + 1 < CHUNKS else None
        for cp in pending:
            cp.wait()
        slot = c % 2

        def group(g, carry):
            r0 = g * 16
            hv16 = ch_v[c, pl.ds(r0, 16)]
            hu16 = xh_v[c, pl.ds(r0, 16)]
            acc = jnp.zeros((16,), jnp.float32)
            for j in range(16):
                r = r0 + j
                jj = jnp.full((16,), j, jnp.int32)
                hv = hv16.at[jj].get(mode="promise_in_bounds")
                hu = hu16.at[jj].get(mode="promise_in_bounds")
                p = jnp.zeros((16,), jnp.float32)
                for k in range(4):
                    alo = vbuf[slot, r, pl.ds(k * 16, 16)]
                    a = alo + hv * (vbuf[slot, r, pl.ds(DIM + k * 16, 16)] - alo)
                    blo = ubuf[slot, r, pl.ds(k * 16, 16)]
                    b = blo + hu * (ubuf[slot, r, pl.ds(DIM + k * 16, 16)] - blo)
                    p = p + a * b
                for sh in (1, 2, 4, 8):
                    p = p + p.at[lanes ^ sh].get(mode="promise_in_bounds")
                acc = jnp.where(lanes == j, p, acc)
            out_v[pl.ds(c * CHUNK + r0, 16)] = acc
            return carry

        lax.fori_loop(0, GROUPS, group, 0)
        pending = nxt

    pltpu.sync_copy(out_v, out_hbm.at[pl.ds(base, BPW)])


def kernel(center_idx, context_idx, in_embed, out_embed):
    pv = _pack(in_embed.T)
    pu = _pack(out_embed.T)
    cidx = center_idx.astype(jnp.int32)
    xidx = context_idx.astype(jnp.int32)
    cq = jnp.where(cidx >= S, cidx - S, cidx).reshape(NUM_WORKERS, CHUNKS, CHUNK)
    xq = jnp.where(xidx >= S, xidx - S, xidx).reshape(NUM_WORKERS, CHUNKS, CHUNK)
    ch = (cidx >= S).astype(jnp.float32).reshape(NUM_WORKERS, CHUNKS, CHUNK)
    xh = (xidx >= S).astype(jnp.float32).reshape(NUM_WORKERS, CHUNKS, CHUNK)
    return _w2v_dot(pv, pu, cq, xq, ch, xh)
